# baseline (speedup 1.0000x reference)
import math

import jax
import jax.numpy as jnp
import numpy as np
from jax.experimental import pallas as pl

D_MODEL = 1024
SEQ_LEN = 4096
BATCH = 4
SEQ_TILE = 256
QBLK = SEQ_TILE // 64


def _pe_tables():
    d = np.arange(D_MODEL)
    w = np.exp((d - (d % 2)).astype(np.float64) * (-math.log(10000.0) / D_MODEL))
    phi = (d % 2).astype(np.float64) * (math.pi / 2)
    q = np.arange(64, dtype=np.float64)[:, None]
    r = np.arange(64, dtype=np.float64)[:, None]
    P = np.sin(64.0 * q * w + phi)
    Q = np.cos(64.0 * q * w + phi)
    Bc = np.cos(r * w)
    Bs = np.sin(r * w)
    pq = np.stack([P, Q], axis=1).astype(np.float32)
    b = np.stack([Bc, Bs]).astype(np.float32)
    return pq, b


def _body(x_ref, f_ref, me_ref, pq_ref, b_ref, o_ref):
    Bc = b_ref[0]
    Bs = b_ref[1]
    me0 = me_ref[0:1, :]
    dme = me_ref[1:2, :] - me0
    for j in range(QBLK):
        P = pq_ref[j : j + 1, 0, :]
        Q = pq_ref[j : j + 1, 1, :]
        rows = pl.ds(64 * j, 64)
        add = P * Bc + Q * Bs + me0 + f_ref[rows, :] * dme
        o_ref[:, rows, :] = x_ref[:, rows, :] + add[None]


def kernel(x, mask_embed, mask_idx):
    pq_np, b_np = _pe_tables()
    pq = jnp.asarray(pq_np)
    b = jnp.asarray(b_np)
    f = mask_idx.astype(jnp.float32).reshape(SEQ_LEN, 1)
    grid = (SEQ_LEN // SEQ_TILE,)
    return pl.pallas_call(
        _body,
        grid=grid,
        in_specs=[
            pl.BlockSpec((BATCH, SEQ_TILE, D_MODEL), lambda i: (0, i, 0)),
            pl.BlockSpec((SEQ_TILE, 1), lambda i: (i, 0)),
            pl.BlockSpec((2, D_MODEL), lambda i: (0, 0)),
            pl.BlockSpec((QBLK, 2, D_MODEL), lambda i: (i, 0, 0)),
            pl.BlockSpec((2, 64, D_MODEL), lambda i: (0, 0, 0)),
        ],
        out_specs=pl.BlockSpec((BATCH, SEQ_TILE, D_MODEL), lambda i: (0, i, 0)),
        out_shape=jax.ShapeDtypeStruct((BATCH, SEQ_LEN, D_MODEL), jnp.float32),
    )(x, f, mask_embed, pq, b)


# speedup vs baseline: 1.0241x; 1.0241x over previous
import math

import jax
import jax.numpy as jnp
import numpy as np
from jax.experimental import pallas as pl

D_MODEL = 1024
SEQ_LEN = 4096
BATCH = 4
SEQ_TILE = 512
SLAB = 128
QBLK = SEQ_TILE // SLAB


def _pe_tables():
    d = np.arange(D_MODEL)
    w = np.exp((d - (d % 2)).astype(np.float64) * (-math.log(10000.0) / D_MODEL))
    phi = (d % 2).astype(np.float64) * (math.pi / 2)
    q = np.arange(SEQ_LEN // SLAB, dtype=np.float64)[:, None]
    r = np.arange(SLAB, dtype=np.float64)[:, None]
    P = np.sin(float(SLAB) * q * w + phi)
    Q = np.cos(float(SLAB) * q * w + phi)
    Bc = np.cos(r * w)
    Bs = np.sin(r * w)
    pq = np.stack([P, Q], axis=1).astype(np.float32)
    b = np.stack([Bc, Bs]).astype(np.float32)
    return pq, b


def _body(x_ref, f_ref, me_ref, pq_ref, b_ref, o_ref):
    Bc = b_ref[0]
    Bs = b_ref[1]
    me0 = me_ref[0:1, :]
    dme = me_ref[1:2, :] - me0
    for j in range(QBLK):
        P = pq_ref[j : j + 1, 0, :]
        Q = pq_ref[j : j + 1, 1, :]
        rows = pl.ds(SLAB * j, SLAB)
        add = P * Bc + Q * Bs + me0 + f_ref[rows, :] * dme
        o_ref[:, rows, :] = x_ref[:, rows, :] + add[None]


def kernel(x, mask_embed, mask_idx):
    pq_np, b_np = _pe_tables()
    pq = jnp.asarray(pq_np)
    b = jnp.asarray(b_np)
    f = mask_idx.astype(jnp.float32).reshape(SEQ_LEN, 1)
    grid = (SEQ_LEN // SEQ_TILE,)
    return pl.pallas_call(
        _body,
        grid=grid,
        in_specs=[
            pl.BlockSpec((BATCH, SEQ_TILE, D_MODEL), lambda i: (0, i, 0)),
            pl.BlockSpec((SEQ_TILE, 1), lambda i: (i, 0)),
            pl.BlockSpec((2, D_MODEL), lambda i: (0, 0)),
            pl.BlockSpec((QBLK, 2, D_MODEL), lambda i: (i, 0, 0)),
            pl.BlockSpec((2, SLAB, D_MODEL), lambda i: (0, 0, 0)),
        ],
        out_specs=pl.BlockSpec((BATCH, SEQ_TILE, D_MODEL), lambda i: (0, i, 0)),
        out_shape=jax.ShapeDtypeStruct((BATCH, SEQ_LEN, D_MODEL), jnp.float32),
    )(x, f, mask_embed, pq, b)


# speedup vs baseline: 1.0344x; 1.0101x over previous
import math

import jax
import jax.numpy as jnp
import numpy as np
from jax.experimental import pallas as pl

D_MODEL = 1024
SEQ_LEN = 4096
BATCH = 4
SEQ_TILE = 512
SLAB = 64
QBLK = SEQ_TILE // SLAB


def _pe_tables():
    d = np.arange(D_MODEL)
    w = np.exp((d - (d % 2)).astype(np.float64) * (-math.log(10000.0) / D_MODEL))
    phi = (d % 2).astype(np.float64) * (math.pi / 2)
    q = np.arange(SEQ_LEN // SLAB, dtype=np.float64)[:, None]
    r = np.arange(SLAB, dtype=np.float64)[:, None]
    P = np.sin(float(SLAB) * q * w + phi)
    Q = np.cos(float(SLAB) * q * w + phi)
    Bc = np.cos(r * w)
    Bs = np.sin(r * w)
    pq = np.stack([P, Q], axis=1).astype(np.float32)
    b = np.stack([Bc, Bs]).astype(np.float32)
    return pq, b


def _body(x_ref, f_ref, me_ref, pq_ref, b_ref, o_ref):
    Bc = b_ref[0]
    Bs = b_ref[1]
    me0 = me_ref[0:1, :]
    me1 = me_ref[1:2, :]
    for j in range(QBLK):
        P = pq_ref[j : j + 1, 0, :]
        Q = pq_ref[j : j + 1, 1, :]
        rows = pl.ds(SLAB * j, SLAB)
        mrow = jnp.where(f_ref[rows, :] != 0.0, me1, me0)
        add = P * Bc + Q * Bs + mrow
        o_ref[:, rows, :] = x_ref[:, rows, :] + add[None]


def kernel(x, mask_embed, mask_idx):
    pq_np, b_np = _pe_tables()
    pq = jnp.asarray(pq_np)
    b = jnp.asarray(b_np)
    f = mask_idx.astype(jnp.float32).reshape(SEQ_LEN, 1)
    grid = (SEQ_LEN // SEQ_TILE,)
    return pl.pallas_call(
        _body,
        grid=grid,
        in_specs=[
            pl.BlockSpec((BATCH, SEQ_TILE, D_MODEL), lambda i: (0, i, 0)),
            pl.BlockSpec((SEQ_TILE, 1), lambda i: (i, 0)),
            pl.BlockSpec((2, D_MODEL), lambda i: (0, 0)),
            pl.BlockSpec((QBLK, 2, D_MODEL), lambda i: (i, 0, 0)),
            pl.BlockSpec((2, SLAB, D_MODEL), lambda i: (0, 0, 0)),
        ],
        out_specs=pl.BlockSpec((BATCH, SEQ_TILE, D_MODEL), lambda i: (0, i, 0)),
        out_shape=jax.ShapeDtypeStruct((BATCH, SEQ_LEN, D_MODEL), jnp.float32),
    )(x, f, mask_embed, pq, b)
